# baseline (speedup 1.0000x reference)
import functools

import jax
import jax.numpy as jnp
from jax import lax
from jax.experimental import pallas as pl
from jax.experimental.pallas import tpu as pltpu
from jax.experimental.pallas import tpu_sc as plsc

_FEAT = 128
_NTILES = 16
_EB = 80
_GB = 8
_NBUF = 4


def _round_up(v, m):
    return (v + m - 1) // m * m


def _make_sc_agg(cin, n, n_acc, tile_rows, n_batches):
    n_pairs = cin // 2
    n_groups = n_batches // _GB
    n_blocks = n_batches // (2 * _GB)
    mesh = plsc.VectorSubcoreMesh(core_axis_name="c", subcore_axis_name="s")

    @functools.partial(
        pl.kernel,
        mesh=mesh,
        out_type=jax.ShapeDtypeStruct((cin, n_acc, _FEAT), jnp.float32),
        scratch_types=[
            pltpu.VMEM((2, _GB, _EB), jnp.int32),
            pltpu.VMEM((2, _GB, _EB), jnp.int32),
            pltpu.VMEM((_NBUF, _EB, _FEAT), jnp.float32),
            pltpu.VMEM_SHARED((n_acc, _FEAT), jnp.float32),
            pltpu.SemaphoreType.DMA,
            pltpu.SemaphoreType.DMA,
        ],
    )
    def k(tab, srcr, dstr, zer, out, sidx, didx, rwb, acc, semg, semi):
        c = lax.axis_index("c")
        s = lax.axis_index("s")
        rows = tuple(rwb.at[i] for i in range(_NBUF))

        def idx_pf(goff, p):
            pltpu.async_copy(srcr.at[s].at[pl.ds(goff, _GB)], sidx.at[p],
                             semi)
            pltpu.async_copy(dstr.at[s].at[pl.ds(goff, _GB)], didx.at[p],
                             semi)

        def idx_pf_wait(goff, p):
            pltpu.make_async_copy(srcr.at[s].at[pl.ds(goff, _GB)],
                                  sidx.at[p], semi).wait()
            pltpu.make_async_copy(dstr.at[s].at[pl.ds(goff, _GB)],
                                  didx.at[p], semi).wait()

        def do_block(chunk, g0, full):
            for b in range(16):
                p, rw = b // 8, b % 8
                if b == 0:
                    idx_pf((g0 + 1) * _GB, 1)
                if b == 8 and full:
                    idx_pf((g0 + 2) * _GB, 0)
                if b == 5:
                    idx_pf_wait((g0 + 1) * _GB, 1)
                if b == 13 and full:
                    idx_pf_wait((g0 + 2) * _GB, 0)
                pltpu.make_async_copy(tab.at[chunk].at[sidx.at[p].at[rw]],
                                      rows[b % _NBUF], semg).wait()
                if full or b < 13:
                    if b < 5:
                        np_, nr = 0, b + 3
                    elif b < 13:
                        np_, nr = 1, b - 5
                    else:
                        np_, nr = 0, b - 13
                    pltpu.async_copy(
                        tab.at[chunk].at[sidx.at[np_].at[nr]],
                        rows[(b + 3) % _NBUF], semg)
                pltpu.sync_copy(rows[b % _NBUF], acc.at[didx.at[p].at[rw]],
                                add=True)

        for r in range(n_pairs):
            chunk = 2 * r + c
            row0 = s * tile_rows
            pltpu.sync_copy(srcr.at[s].at[pl.ds(0, _GB)], sidx.at[0])
            pltpu.sync_copy(dstr.at[s].at[pl.ds(0, _GB)], didx.at[0])
            for i in range(3):
                pltpu.async_copy(tab.at[chunk].at[sidx.at[0].at[i]],
                                 rows[i], semg)
            pltpu.sync_copy(zer.at[pl.ds(row0, tile_rows)],
                            acc.at[pl.ds(row0, tile_rows)])
            plsc.subcore_barrier()

            def block_body(t, carry):
                do_block(chunk, 2 * t, True)
                return carry

            lax.fori_loop(0, n_blocks - 1, block_body, 0)
            do_block(chunk, 2 * (n_blocks - 1), False)

            plsc.subcore_barrier()
            pltpu.sync_copy(acc.at[pl.ds(row0, tile_rows)],
                            out.at[chunk].at[pl.ds(row0, tile_rows)])

    return k


def _mlp_call(h, agg, scale, wa, ba, wb, bb, nb):
    cin, n, feat = h.shape
    hdim = wa.shape[1]
    cout = wb.shape[1] // feat
    grid = (n // nb,)

    def body(h_ref, a_ref, sc_ref, wa_ref, ba_ref, wb_ref, bb_ref, out_ref):
        scale_v = sc_ref[0, 0]
        acc = jnp.zeros((nb, hdim), jnp.float32)
        for ci in range(cin):
            z = (scale_v * h_ref[ci].astype(jnp.float32)
                 + a_ref[ci].astype(jnp.float32)).astype(jnp.bfloat16)
            acc = acc + lax.dot_general(
                z, wa_ref[ci * feat:(ci + 1) * feat, :],
                (((1,), (0,)), ((), ())), preferred_element_type=jnp.float32)
        a1 = jnp.maximum(acc + ba_ref[0:1, :], 0.0).astype(jnp.bfloat16)
        h2 = lax.dot_general(a1, wb_ref[...], (((1,), (0,)), ((), ())),
                             preferred_element_type=jnp.float32)
        h2 = jnp.maximum(h2 + bb_ref[0:1, :], 0.0)
        for co in range(cout):
            out_ref[co] = h2[:, co * feat:(co + 1) * feat]

    return pl.pallas_call(
        body,
        grid=grid,
        in_specs=[
            pl.BlockSpec((cin, nb, feat), lambda i: (0, i, 0)),
            pl.BlockSpec((cin, nb, feat), lambda i: (0, i, 0)),
            pl.BlockSpec((1, 1), lambda i: (0, 0)),
            pl.BlockSpec(wa.shape, lambda i: (0, 0)),
            pl.BlockSpec((1, hdim), lambda i: (0, 0)),
            pl.BlockSpec(wb.shape, lambda i: (0, 0)),
            pl.BlockSpec((1, wb.shape[1]), lambda i: (0, 0)),
        ],
        out_specs=pl.BlockSpec((cout, nb, feat), lambda i: (0, i, 0)),
        out_shape=jax.ShapeDtypeStruct((cout, n, feat), jnp.float32),
    )(h, agg, scale, wa, ba, wb, bb)


def _mlp_a_call(h, agg, scale, wa, nb):
    cin, n, feat = h.shape
    hdim = wa.shape[1]

    def body(h_ref, a_ref, sc_ref, wa_ref, out_ref):
        scale_v = sc_ref[0, 0]
        acc = jnp.zeros((nb, hdim), jnp.float32)
        for ci in range(cin):
            z = (scale_v * h_ref[ci] + a_ref[ci]).astype(jnp.bfloat16)
            acc = acc + lax.dot_general(
                z, wa_ref[ci * feat:(ci + 1) * feat, :],
                (((1,), (0,)), ((), ())), preferred_element_type=jnp.float32)
        out_ref[...] = acc.astype(jnp.bfloat16)

    return pl.pallas_call(
        body,
        grid=(n // nb,),
        in_specs=[
            pl.BlockSpec((cin, nb, feat), lambda i: (0, i, 0)),
            pl.BlockSpec((cin, nb, feat), lambda i: (0, i, 0)),
            pl.BlockSpec((1, 1), lambda i: (0, 0)),
            pl.BlockSpec(wa.shape, lambda i: (0, 0)),
        ],
        out_specs=pl.BlockSpec((nb, hdim), lambda i: (i, 0)),
        out_shape=jax.ShapeDtypeStruct((n, hdim), jnp.bfloat16),
    )(h, agg, scale, wa)


def _mlp_b_call(h, agg, part, scale, wa, ba, wb, bb, nb):
    cin, n, feat = h.shape
    hdim = wa.shape[1]
    cout = wb.shape[1] // feat

    def body(h_ref, a_ref, p_ref, sc_ref, wa_ref, ba_ref, wb_ref, bb_ref,
             out_ref):
        scale_v = sc_ref[0, 0]
        acc = p_ref[...].astype(jnp.float32)
        for ci in range(cin):
            z = (scale_v * h_ref[ci] + a_ref[ci]).astype(jnp.bfloat16)
            acc = acc + lax.dot_general(
                z, wa_ref[ci * feat:(ci + 1) * feat, :],
                (((1,), (0,)), ((), ())), preferred_element_type=jnp.float32)
        a1 = jnp.maximum(acc + ba_ref[0:1, :], 0.0).astype(jnp.bfloat16)
        h2 = lax.dot_general(a1, wb_ref[...], (((1,), (0,)), ((), ())),
                             preferred_element_type=jnp.float32)
        h2 = jnp.maximum(h2 + bb_ref[0:1, :], 0.0)
        for co in range(cout):
            out_ref[co] = h2[:, co * feat:(co + 1) * feat]

    return pl.pallas_call(
        body,
        grid=(n // nb,),
        in_specs=[
            pl.BlockSpec((cin, nb, feat), lambda i: (0, i, 0)),
            pl.BlockSpec((cin, nb, feat), lambda i: (0, i, 0)),
            pl.BlockSpec((nb, hdim), lambda i: (i, 0)),
            pl.BlockSpec((1, 1), lambda i: (0, 0)),
            pl.BlockSpec(wa.shape, lambda i: (0, 0)),
            pl.BlockSpec((1, hdim), lambda i: (0, 0)),
            pl.BlockSpec(wb.shape, lambda i: (0, 0)),
            pl.BlockSpec((1, wb.shape[1]), lambda i: (0, 0)),
        ],
        out_specs=pl.BlockSpec((cout, nb, feat), lambda i: (0, i, 0)),
        out_shape=jax.ShapeDtypeStruct((cout, n, feat), jnp.float32),
    )(h, agg, part, scale, wa, ba, wb, bb)


def _pool_cls_call(h, batchr, wc1, bc1, wc2, bc2, nbp, g):
    cin, n, feat = h.shape
    hdim = wc1.shape[1]
    ncls = wc2.shape[1]
    t = n // nbp

    def body(h_ref, b_ref, wc1_ref, bc1_ref, wc2_ref, bc2_ref, out_ref,
             sums, cnt):
        i = pl.program_id(0)

        @pl.when(i == 0)
        def _():
            sums[...] = jnp.zeros_like(sums)
            cnt[...] = jnp.zeros_like(cnt)

        b2 = b_ref[0]
        iota_g = lax.broadcasted_iota(jnp.int32, (g, nbp), 0)
        oht = (b2 == iota_g).astype(jnp.bfloat16)
        for ci in range(cin):
            sums[ci] += lax.dot_general(
                oht, h_ref[ci].astype(jnp.bfloat16), (((1,), (0,)), ((), ())),
                preferred_element_type=jnp.float32)
        cnt[...] += lax.dot_general(
            oht, jnp.ones((nbp, feat), jnp.bfloat16),
            (((1,), (0,)), ((), ())), preferred_element_type=jnp.float32)

        @pl.when(i == t - 1)
        def _():
            rcp = 1.0 / jnp.maximum(cnt[...], 1.0)
            acc = jnp.zeros((g, hdim), jnp.float32)
            for ci in range(cin):
                pooled = (sums[ci] * rcp).astype(jnp.bfloat16)
                acc = acc + lax.dot_general(
                    pooled, wc1_ref[ci * feat:(ci + 1) * feat, :],
                    (((1,), (0,)), ((), ())),
                    preferred_element_type=jnp.float32)
            hc = jnp.maximum(acc + bc1_ref[0:1, :], 0.0).astype(jnp.bfloat16)
            logits = lax.dot_general(hc, wc2_ref[...],
                                     (((1,), (0,)), ((), ())),
                                     preferred_element_type=jnp.float32)
            out_ref[...] = logits + bc2_ref[0:1, :]

    return pl.pallas_call(
        body,
        grid=(t,),
        in_specs=[
            pl.BlockSpec((cin, nbp, feat), lambda i: (0, i, 0)),
            pl.BlockSpec((1, 1, nbp), lambda i: (i, 0, 0)),
            pl.BlockSpec(wc1.shape, lambda i: (0, 0)),
            pl.BlockSpec((1, hdim), lambda i: (0, 0)),
            pl.BlockSpec(wc2.shape, lambda i: (0, 0)),
            pl.BlockSpec((1, ncls), lambda i: (0, 0)),
        ],
        out_specs=pl.BlockSpec((g, ncls), lambda i: (0, 0)),
        out_shape=jax.ShapeDtypeStruct((g, ncls), jnp.float32),
        scratch_shapes=[
            pltpu.VMEM((cin, g, feat), jnp.float32),
            pltpu.VMEM((g, feat), jnp.float32),
        ],
    )(h, batchr, wc1, bc1, wc2, bc2)


def kernel(x, edge_index, batch, eps0, W0a, b0a, W0b, b0b, eps1, W1a, b1a,
           W1b, b1b, eps2, W2a, b2a, W2b, b2b, Wc1, bc1, Wc2, bc2):
    n, in_c = x.shape
    e = edge_index.shape[1]
    g = 128

    tile_rows = _round_up(n // _NTILES + 7, 8)
    n_acc = tile_rows * _NTILES

    ept = -(-e // _NTILES)
    n_batches = _round_up(-(-ept // _EB), 2 * _GB)
    e_pad = _NTILES * n_batches * _EB
    pad = e_pad - e
    src = edge_index[0]
    dst = edge_index[1]
    ar = jnp.arange(pad, dtype=jnp.int32)
    srcp = jnp.concatenate([src, ar % n]).reshape(_NTILES, n_batches, _EB)
    dstp = jnp.concatenate([dst, n + ar % (n_acc - n)]
                           ).reshape(_NTILES, n_batches, _EB)
    zeros = jnp.zeros((n_acc, _FEAT), jnp.float32)

    h = jnp.moveaxis(x.reshape(n, in_c // _FEAT, _FEAT), 1, 0)

    layers = ((eps0, W0a, b0a, W0b, b0b), (eps1, W1a, b1a, W1b, b1b),
              (eps2, W2a, b2a, W2b, b2b))
    sc2 = None
    for eps, wa, ba, wb, bb in layers:
        cin = h.shape[0]
        scale = (1.0 + eps).astype(jnp.float32).reshape(1, 1)
        wa16 = wa.astype(jnp.bfloat16)
        wb16 = wb.astype(jnp.bfloat16)
        if cin == 2:
            agg_fn = _make_sc_agg(cin, n, n_acc, tile_rows, n_batches)
            agg = agg_fn(h, srcp, dstp, zeros)
            h = _mlp_call(h, agg, scale, wa16, ba.reshape(1, -1), wb16,
                          bb.reshape(1, -1), nb=1000)
        else:
            if sc2 is None:
                sc2 = _make_sc_agg(2, n, n_acc, tile_rows, n_batches)
            agg_a = sc2(h[0:2], srcp, dstp, zeros)
            agg_b = sc2(h[2:4], srcp, dstp, zeros)
            part = _mlp_a_call(h[0:2], agg_a, scale, wa16[:2 * _FEAT],
                               nb=1000)
            h = _mlp_b_call(h[2:4], agg_b, part, scale, wa16[2 * _FEAT:],
                            ba.reshape(1, -1), wb16, bb.reshape(1, -1),
                            nb=1000)

    nbp = 1000
    batchr = batch.reshape(n // nbp, 1, nbp)
    return _pool_cls_call(h, batchr, Wc1.astype(jnp.bfloat16),
                          bc1.reshape(1, -1), Wc2.astype(jnp.bfloat16),
                          bc2.reshape(1, -1), nbp, g)


# speedup vs baseline: 1.0671x; 1.0671x over previous
import functools

import jax
import jax.numpy as jnp
from jax import lax
from jax.experimental import pallas as pl
from jax.experimental.pallas import tpu as pltpu
from jax.experimental.pallas import tpu_sc as plsc

_FEAT = 128
_NTILES = 16
_EB = 80
_GB = 8
_NBUF = 4


def _round_up(v, m):
    return (v + m - 1) // m * m


def _make_sc_agg(cin, n, n_acc, tile_rows, n_batches):
    n_pairs = cin // 2
    n_groups = n_batches // _GB
    n_blocks = n_batches // (2 * _GB)
    mesh = plsc.VectorSubcoreMesh(core_axis_name="c", subcore_axis_name="s")

    @functools.partial(
        pl.kernel,
        mesh=mesh,
        out_type=jax.ShapeDtypeStruct((cin, n_acc, _FEAT), jnp.float32),
        scratch_types=[
            pltpu.VMEM((2, _GB, _EB), jnp.int32),
            pltpu.VMEM((2, _GB, _EB), jnp.int32),
            pltpu.VMEM((_NBUF, _EB, _FEAT), jnp.float32),
            pltpu.VMEM_SHARED((n_acc, _FEAT), jnp.float32),
            pltpu.SemaphoreType.DMA,
            pltpu.SemaphoreType.DMA,
        ],
    )
    def k(tab, srcr, dstr, zer, out, sidx, didx, rwb, acc, semg, semi):
        c = lax.axis_index("c")
        s = lax.axis_index("s")
        rows = tuple(rwb.at[i] for i in range(_NBUF))

        def idx_pf(goff, p):
            pltpu.async_copy(srcr.at[s].at[pl.ds(goff, _GB)], sidx.at[p],
                             semi)
            pltpu.async_copy(dstr.at[s].at[pl.ds(goff, _GB)], didx.at[p],
                             semi)

        def idx_pf_wait(goff, p):
            pltpu.make_async_copy(srcr.at[s].at[pl.ds(goff, _GB)],
                                  sidx.at[p], semi).wait()
            pltpu.make_async_copy(dstr.at[s].at[pl.ds(goff, _GB)],
                                  didx.at[p], semi).wait()

        def do_block(chunk, g0, full):
            for b in range(16):
                p, rw = b // 8, b % 8
                if b == 0:
                    idx_pf((g0 + 1) * _GB, 1)
                if b == 8 and full:
                    idx_pf((g0 + 2) * _GB, 0)
                if b == 5:
                    idx_pf_wait((g0 + 1) * _GB, 1)
                if b == 13 and full:
                    idx_pf_wait((g0 + 2) * _GB, 0)
                pltpu.make_async_copy(tab.at[chunk].at[sidx.at[p].at[rw]],
                                      rows[b % _NBUF], semg).wait()
                if full or b < 13:
                    if b < 5:
                        np_, nr = 0, b + 3
                    elif b < 13:
                        np_, nr = 1, b - 5
                    else:
                        np_, nr = 0, b - 13
                    pltpu.async_copy(
                        tab.at[chunk].at[sidx.at[np_].at[nr]],
                        rows[(b + 3) % _NBUF], semg)
                pltpu.sync_copy(rows[b % _NBUF], acc.at[didx.at[p].at[rw]],
                                add=True)

        for r in range(n_pairs):
            chunk = 2 * r + c
            row0 = s * tile_rows
            pltpu.sync_copy(srcr.at[s].at[pl.ds(0, _GB)], sidx.at[0])
            pltpu.sync_copy(dstr.at[s].at[pl.ds(0, _GB)], didx.at[0])
            for i in range(3):
                pltpu.async_copy(tab.at[chunk].at[sidx.at[0].at[i]],
                                 rows[i], semg)
            pltpu.sync_copy(zer.at[pl.ds(row0, tile_rows)],
                            acc.at[pl.ds(row0, tile_rows)])
            plsc.subcore_barrier()

            def block_body(t, carry):
                do_block(chunk, 2 * t, True)
                return carry

            lax.fori_loop(0, n_blocks - 1, block_body, 0)
            do_block(chunk, 2 * (n_blocks - 1), False)

            plsc.subcore_barrier()
            pltpu.sync_copy(acc.at[pl.ds(row0, tile_rows)],
                            out.at[chunk].at[pl.ds(row0, tile_rows)])

    return k


def _mlp_call(h, agg, scale, wa, ba, wb, bb, nb):
    cin, n, feat = h.shape
    hdim = wa.shape[1]
    cout = wb.shape[1] // feat
    grid = (n // nb,)

    def body(h_ref, a_ref, sc_ref, wa_ref, ba_ref, wb_ref, bb_ref, out_ref):
        scale_v = sc_ref[0, 0]
        acc = jnp.zeros((nb, hdim), jnp.float32)
        for ci in range(cin):
            z = (scale_v * h_ref[ci].astype(jnp.float32)
                 + a_ref[ci].astype(jnp.float32)).astype(jnp.bfloat16)
            acc = acc + lax.dot_general(
                z, wa_ref[ci * feat:(ci + 1) * feat, :],
                (((1,), (0,)), ((), ())), preferred_element_type=jnp.float32)
        a1 = jnp.maximum(acc + ba_ref[0:1, :], 0.0).astype(jnp.bfloat16)
        h2 = lax.dot_general(a1, wb_ref[...], (((1,), (0,)), ((), ())),
                             preferred_element_type=jnp.float32)
        h2 = jnp.maximum(h2 + bb_ref[0:1, :], 0.0)
        for co in range(cout):
            out_ref[co] = h2[:, co * feat:(co + 1) * feat]

    return pl.pallas_call(
        body,
        grid=grid,
        in_specs=[
            pl.BlockSpec((cin, nb, feat), lambda i: (0, i, 0)),
            pl.BlockSpec((cin, nb, feat), lambda i: (0, i, 0)),
            pl.BlockSpec((1, 1), lambda i: (0, 0)),
            pl.BlockSpec(wa.shape, lambda i: (0, 0)),
            pl.BlockSpec((1, hdim), lambda i: (0, 0)),
            pl.BlockSpec(wb.shape, lambda i: (0, 0)),
            pl.BlockSpec((1, wb.shape[1]), lambda i: (0, 0)),
        ],
        out_specs=pl.BlockSpec((cout, nb, feat), lambda i: (0, i, 0)),
        out_shape=jax.ShapeDtypeStruct((cout, n, feat), jnp.float32),
    )(h, agg, scale, wa, ba, wb, bb)


def _pool_cls_call(h, batchr, wc1, bc1, wc2, bc2, nbp, g):
    cin, n, feat = h.shape
    hdim = wc1.shape[1]
    ncls = wc2.shape[1]
    t = n // nbp

    def body(h_ref, b_ref, wc1_ref, bc1_ref, wc2_ref, bc2_ref, out_ref,
             sums, cnt):
        i = pl.program_id(0)

        @pl.when(i == 0)
        def _():
            sums[...] = jnp.zeros_like(sums)
            cnt[...] = jnp.zeros_like(cnt)

        b2 = b_ref[0]
        iota_g = lax.broadcasted_iota(jnp.int32, (g, nbp), 0)
        oht = (b2 == iota_g).astype(jnp.bfloat16)
        for ci in range(cin):
            sums[ci] += lax.dot_general(
                oht, h_ref[ci].astype(jnp.bfloat16), (((1,), (0,)), ((), ())),
                preferred_element_type=jnp.float32)
        cnt[...] += lax.dot_general(
            oht, jnp.ones((nbp, feat), jnp.bfloat16),
            (((1,), (0,)), ((), ())), preferred_element_type=jnp.float32)

        @pl.when(i == t - 1)
        def _():
            rcp = 1.0 / jnp.maximum(cnt[...], 1.0)
            acc = jnp.zeros((g, hdim), jnp.float32)
            for ci in range(cin):
                pooled = (sums[ci] * rcp).astype(jnp.bfloat16)
                acc = acc + lax.dot_general(
                    pooled, wc1_ref[ci * feat:(ci + 1) * feat, :],
                    (((1,), (0,)), ((), ())),
                    preferred_element_type=jnp.float32)
            hc = jnp.maximum(acc + bc1_ref[0:1, :], 0.0).astype(jnp.bfloat16)
            logits = lax.dot_general(hc, wc2_ref[...],
                                     (((1,), (0,)), ((), ())),
                                     preferred_element_type=jnp.float32)
            out_ref[...] = logits + bc2_ref[0:1, :]

    return pl.pallas_call(
        body,
        grid=(t,),
        in_specs=[
            pl.BlockSpec((cin, nbp, feat), lambda i: (0, i, 0)),
            pl.BlockSpec((1, 1, nbp), lambda i: (i, 0, 0)),
            pl.BlockSpec(wc1.shape, lambda i: (0, 0)),
            pl.BlockSpec((1, hdim), lambda i: (0, 0)),
            pl.BlockSpec(wc2.shape, lambda i: (0, 0)),
            pl.BlockSpec((1, ncls), lambda i: (0, 0)),
        ],
        out_specs=pl.BlockSpec((g, ncls), lambda i: (0, 0)),
        out_shape=jax.ShapeDtypeStruct((g, ncls), jnp.float32),
        scratch_shapes=[
            pltpu.VMEM((cin, g, feat), jnp.float32),
            pltpu.VMEM((g, feat), jnp.float32),
        ],
    )(h, batchr, wc1, bc1, wc2, bc2)


def kernel(x, edge_index, batch, eps0, W0a, b0a, W0b, b0b, eps1, W1a, b1a,
           W1b, b1b, eps2, W2a, b2a, W2b, b2b, Wc1, bc1, Wc2, bc2):
    n, in_c = x.shape
    e = edge_index.shape[1]
    g = 128

    tile_rows = _round_up(n // _NTILES + 7, 8)
    n_acc = tile_rows * _NTILES

    ept = -(-e // _NTILES)
    n_batches = _round_up(-(-ept // _EB), 2 * _GB)
    e_pad = _NTILES * n_batches * _EB
    pad = e_pad - e
    src = edge_index[0]
    dst = edge_index[1]
    ar = jnp.arange(pad, dtype=jnp.int32)
    srcp = jnp.concatenate([src, ar % n]).reshape(_NTILES, n_batches, _EB)
    dstp = jnp.concatenate([dst, n + ar % (n_acc - n)]
                           ).reshape(_NTILES, n_batches, _EB)
    zeros = jnp.zeros((n_acc, _FEAT), jnp.float32)

    h = jnp.moveaxis(x.reshape(n, in_c // _FEAT, _FEAT), 1, 0)

    layers = ((eps0, W0a, b0a, W0b, b0b), (eps1, W1a, b1a, W1b, b1b),
              (eps2, W2a, b2a, W2b, b2b))
    for eps, wa, ba, wb, bb in layers:
        cin = h.shape[0]
        agg_fn = _make_sc_agg(cin, n, n_acc, tile_rows, n_batches)
        agg = agg_fn(h, srcp, dstp, zeros)
        scale = (1.0 + eps).astype(jnp.float32).reshape(1, 1)
        h = _mlp_call(h, agg, scale, wa.astype(jnp.bfloat16),
                      ba.reshape(1, -1), wb.astype(jnp.bfloat16),
                      bb.reshape(1, -1), nb=1000)

    nbp = 1000
    batchr = batch.reshape(n // nbp, 1, nbp)
    return _pool_cls_call(h, batchr, Wc1.astype(jnp.bfloat16),
                          bc1.reshape(1, -1), Wc2.astype(jnp.bfloat16),
                          bc2.reshape(1, -1), nbp, g)


# speedup vs baseline: 1.0937x; 1.0250x over previous
import functools

import jax
import jax.numpy as jnp
from jax import lax
from jax.experimental import pallas as pl
from jax.experimental.pallas import tpu as pltpu
from jax.experimental.pallas import tpu_sc as plsc

_FEAT = 128
_NTILES = 16
_EB = 80
_GB = 8
_NBUF = 4


def _round_up(v, m):
    return (v + m - 1) // m * m


def _make_sc_agg(cin, n, n_acc, tile_rows, n_batches):
    n_pairs = cin // 2
    n_groups = n_batches // _GB
    n_blocks = n_batches // (2 * _GB)
    mesh = plsc.VectorSubcoreMesh(core_axis_name="c", subcore_axis_name="s")

    @functools.partial(
        pl.kernel,
        mesh=mesh,
        out_type=jax.ShapeDtypeStruct((cin, n_acc, _FEAT), jnp.float32),
        scratch_types=[
            pltpu.VMEM((2, _GB, _EB), jnp.int32),
            pltpu.VMEM((2, _GB, _EB), jnp.int32),
            pltpu.VMEM((_NBUF, _EB, _FEAT), jnp.float32),
            pltpu.VMEM_SHARED((n_acc, _FEAT), jnp.float32),
            pltpu.SemaphoreType.DMA,
            pltpu.SemaphoreType.DMA,
        ],
    )
    def k(tab, srcr, dstr, zer, out, sidx, didx, rwb, acc, semg, semi):
        c = lax.axis_index("c")
        s = lax.axis_index("s")
        rows = tuple(rwb.at[i] for i in range(_NBUF))

        def idx_pf(goff, p):
            pltpu.async_copy(srcr.at[s].at[pl.ds(goff, _GB)], sidx.at[p],
                             semi)
            pltpu.async_copy(dstr.at[s].at[pl.ds(goff, _GB)], didx.at[p],
                             semi)

        def idx_pf_wait(goff, p):
            pltpu.make_async_copy(srcr.at[s].at[pl.ds(goff, _GB)],
                                  sidx.at[p], semi).wait()
            pltpu.make_async_copy(dstr.at[s].at[pl.ds(goff, _GB)],
                                  didx.at[p], semi).wait()

        def do_block(chunk, g0, full):
            for b in range(16):
                p, rw = b // 8, b % 8
                if b == 0:
                    idx_pf((g0 + 1) * _GB, 1)
                if b == 8 and full:
                    idx_pf((g0 + 2) * _GB, 0)
                if b == 5:
                    idx_pf_wait((g0 + 1) * _GB, 1)
                if b == 13 and full:
                    idx_pf_wait((g0 + 2) * _GB, 0)
                pltpu.make_async_copy(tab.at[chunk].at[sidx.at[p].at[rw]],
                                      rows[b % _NBUF], semg).wait()
                if full or b < 13:
                    if b < 5:
                        np_, nr = 0, b + 3
                    elif b < 13:
                        np_, nr = 1, b - 5
                    else:
                        np_, nr = 0, b - 13
                    pltpu.async_copy(
                        tab.at[chunk].at[sidx.at[np_].at[nr]],
                        rows[(b + 3) % _NBUF], semg)
                pltpu.sync_copy(rows[b % _NBUF], acc.at[didx.at[p].at[rw]],
                                add=True)

        for r in range(n_pairs):
            chunk = 2 * r + c
            row0 = s * tile_rows
            pltpu.sync_copy(srcr.at[s].at[pl.ds(0, _GB)], sidx.at[0])
            pltpu.sync_copy(dstr.at[s].at[pl.ds(0, _GB)], didx.at[0])
            for i in range(3):
                pltpu.async_copy(tab.at[chunk].at[sidx.at[0].at[i]],
                                 rows[i], semg)
            pltpu.sync_copy(zer.at[pl.ds(row0, tile_rows)],
                            acc.at[pl.ds(row0, tile_rows)])
            plsc.subcore_barrier()

            def block_body(t, carry):
                do_block(chunk, 2 * t, True)
                return carry

            lax.fori_loop(0, n_blocks - 1, block_body, 0)
            do_block(chunk, 2 * (n_blocks - 1), False)

            plsc.subcore_barrier()
            pltpu.sync_copy(acc.at[pl.ds(row0, tile_rows)],
                            out.at[chunk].at[pl.ds(row0, tile_rows)])

    return k


def _mlp_call(h, agg, scale, wa, ba, wb, bb, nb):
    cin, n, feat = h.shape
    hdim = wa.shape[1]
    cout = wb.shape[1] // feat
    grid = (n // nb,)

    def body(h_ref, a_ref, sc_ref, wa_ref, ba_ref, wb_ref, bb_ref, out_ref):
        scale_v = sc_ref[0, 0]
        acc = jnp.zeros((nb, hdim), jnp.float32)
        for ci in range(cin):
            z = (scale_v * h_ref[ci].astype(jnp.float32)
                 + a_ref[ci].astype(jnp.float32)).astype(jnp.bfloat16)
            acc = acc + lax.dot_general(
                z, wa_ref[ci * feat:(ci + 1) * feat, :],
                (((1,), (0,)), ((), ())), preferred_element_type=jnp.float32)
        a1 = jnp.maximum(acc + ba_ref[0:1, :], 0.0).astype(jnp.bfloat16)
        h2 = lax.dot_general(a1, wb_ref[...], (((1,), (0,)), ((), ())),
                             preferred_element_type=jnp.float32)
        h2 = jnp.maximum(h2 + bb_ref[0:1, :], 0.0)
        for co in range(cout):
            out_ref[co] = h2[:, co * feat:(co + 1) * feat]

    return pl.pallas_call(
        body,
        grid=grid,
        in_specs=[
            pl.BlockSpec((cin, nb, feat), lambda i: (0, i, 0)),
            pl.BlockSpec((cin, nb, feat), lambda i: (0, i, 0)),
            pl.BlockSpec((1, 1), lambda i: (0, 0)),
            pl.BlockSpec(wa.shape, lambda i: (0, 0)),
            pl.BlockSpec((1, hdim), lambda i: (0, 0)),
            pl.BlockSpec(wb.shape, lambda i: (0, 0)),
            pl.BlockSpec((1, wb.shape[1]), lambda i: (0, 0)),
        ],
        out_specs=pl.BlockSpec((cout, nb, feat), lambda i: (0, i, 0)),
        out_shape=jax.ShapeDtypeStruct((cout, n, feat), jnp.float32),
    )(h, agg, scale, wa, ba, wb, bb)


def _mlp_pool_cls_call(h, agg, scale, wa, ba, wb, bb, batchr, wc1, bc1,
                       wc2, bc2, nb, g):
    cin, n, feat = h.shape
    hdim = wa.shape[1]
    ncls = wc2.shape[1]
    t = n // nb

    def body(h_ref, a_ref, b_ref, sc_ref, wa_ref, ba_ref, wb_ref, bb_ref,
             wc1_ref, bc1_ref, wc2_ref, bc2_ref, out_ref, sums, cnt):
        i = pl.program_id(0)

        @pl.when(i == 0)
        def _():
            sums[...] = jnp.zeros_like(sums)
            cnt[...] = jnp.zeros_like(cnt)

        scale_v = sc_ref[0, 0]
        acc = jnp.zeros((nb, hdim), jnp.float32)
        for ci in range(cin):
            z = (scale_v * h_ref[ci] + a_ref[ci]).astype(jnp.bfloat16)
            acc = acc + lax.dot_general(
                z, wa_ref[ci * feat:(ci + 1) * feat, :],
                (((1,), (0,)), ((), ())), preferred_element_type=jnp.float32)
        a1 = jnp.maximum(acc + ba_ref[0:1, :], 0.0).astype(jnp.bfloat16)
        h2 = lax.dot_general(a1, wb_ref[...], (((1,), (0,)), ((), ())),
                             preferred_element_type=jnp.float32)
        h2 = jnp.maximum(h2 + bb_ref[0:1, :], 0.0).astype(jnp.bfloat16)

        b2 = b_ref[0]
        iota_g = lax.broadcasted_iota(jnp.int32, (g, nb), 0)
        oht = (b2 == iota_g).astype(jnp.bfloat16)
        sums[...] += lax.dot_general(oht, h2, (((1,), (0,)), ((), ())),
                                     preferred_element_type=jnp.float32)
        cnt[...] += lax.dot_general(
            oht, jnp.ones((nb, feat), jnp.bfloat16),
            (((1,), (0,)), ((), ())), preferred_element_type=jnp.float32)

        @pl.when(i == t - 1)
        def _():
            rcp = 1.0 / jnp.maximum(cnt[...], 1.0)
            ac2 = jnp.zeros((g, hdim), jnp.float32)
            for ci in range(hdim // feat):
                pooled = (sums[:, ci * feat:(ci + 1) * feat] * rcp
                          ).astype(jnp.bfloat16)
                ac2 = ac2 + lax.dot_general(
                    pooled, wc1_ref[ci * feat:(ci + 1) * feat, :],
                    (((1,), (0,)), ((), ())),
                    preferred_element_type=jnp.float32)
            hc = jnp.maximum(ac2 + bc1_ref[0:1, :], 0.0).astype(jnp.bfloat16)
            logits = lax.dot_general(hc, wc2_ref[...],
                                     (((1,), (0,)), ((), ())),
                                     preferred_element_type=jnp.float32)
            out_ref[...] = logits + bc2_ref[0:1, :]

    return pl.pallas_call(
        body,
        grid=(t,),
        in_specs=[
            pl.BlockSpec((cin, nb, feat), lambda i: (0, i, 0)),
            pl.BlockSpec((cin, nb, feat), lambda i: (0, i, 0)),
            pl.BlockSpec((1, 1, nb), lambda i: (i, 0, 0)),
            pl.BlockSpec((1, 1), lambda i: (0, 0)),
            pl.BlockSpec(wa.shape, lambda i: (0, 0)),
            pl.BlockSpec((1, hdim), lambda i: (0, 0)),
            pl.BlockSpec(wb.shape, lambda i: (0, 0)),
            pl.BlockSpec((1, wb.shape[1]), lambda i: (0, 0)),
            pl.BlockSpec(wc1.shape, lambda i: (0, 0)),
            pl.BlockSpec((1, wc1.shape[1]), lambda i: (0, 0)),
            pl.BlockSpec(wc2.shape, lambda i: (0, 0)),
            pl.BlockSpec((1, ncls), lambda i: (0, 0)),
        ],
        out_specs=pl.BlockSpec((g, ncls), lambda i: (0, 0)),
        out_shape=jax.ShapeDtypeStruct((g, ncls), jnp.float32),
        scratch_shapes=[
            pltpu.VMEM((g, hdim), jnp.float32),
            pltpu.VMEM((g, feat), jnp.float32),
        ],
    )(h, agg, batchr, scale, wa, ba, wb, bb, wc1, bc1, wc2, bc2)


def _pool_cls_call(h, batchr, wc1, bc1, wc2, bc2, nbp, g):
    cin, n, feat = h.shape
    hdim = wc1.shape[1]
    ncls = wc2.shape[1]
    t = n // nbp

    def body(h_ref, b_ref, wc1_ref, bc1_ref, wc2_ref, bc2_ref, out_ref,
             sums, cnt):
        i = pl.program_id(0)

        @pl.when(i == 0)
        def _():
            sums[...] = jnp.zeros_like(sums)
            cnt[...] = jnp.zeros_like(cnt)

        b2 = b_ref[0]
        iota_g = lax.broadcasted_iota(jnp.int32, (g, nbp), 0)
        oht = (b2 == iota_g).astype(jnp.bfloat16)
        for ci in range(cin):
            sums[ci] += lax.dot_general(
                oht, h_ref[ci].astype(jnp.bfloat16), (((1,), (0,)), ((), ())),
                preferred_element_type=jnp.float32)
        cnt[...] += lax.dot_general(
            oht, jnp.ones((nbp, feat), jnp.bfloat16),
            (((1,), (0,)), ((), ())), preferred_element_type=jnp.float32)

        @pl.when(i == t - 1)
        def _():
            rcp = 1.0 / jnp.maximum(cnt[...], 1.0)
            acc = jnp.zeros((g, hdim), jnp.float32)
            for ci in range(cin):
                pooled = (sums[ci] * rcp).astype(jnp.bfloat16)
                acc = acc + lax.dot_general(
                    pooled, wc1_ref[ci * feat:(ci + 1) * feat, :],
                    (((1,), (0,)), ((), ())),
                    preferred_element_type=jnp.float32)
            hc = jnp.maximum(acc + bc1_ref[0:1, :], 0.0).astype(jnp.bfloat16)
            logits = lax.dot_general(hc, wc2_ref[...],
                                     (((1,), (0,)), ((), ())),
                                     preferred_element_type=jnp.float32)
            out_ref[...] = logits + bc2_ref[0:1, :]

    return pl.pallas_call(
        body,
        grid=(t,),
        in_specs=[
            pl.BlockSpec((cin, nbp, feat), lambda i: (0, i, 0)),
            pl.BlockSpec((1, 1, nbp), lambda i: (i, 0, 0)),
            pl.BlockSpec(wc1.shape, lambda i: (0, 0)),
            pl.BlockSpec((1, hdim), lambda i: (0, 0)),
            pl.BlockSpec(wc2.shape, lambda i: (0, 0)),
            pl.BlockSpec((1, ncls), lambda i: (0, 0)),
        ],
        out_specs=pl.BlockSpec((g, ncls), lambda i: (0, 0)),
        out_shape=jax.ShapeDtypeStruct((g, ncls), jnp.float32),
        scratch_shapes=[
            pltpu.VMEM((cin, g, feat), jnp.float32),
            pltpu.VMEM((g, feat), jnp.float32),
        ],
    )(h, batchr, wc1, bc1, wc2, bc2)


def kernel(x, edge_index, batch, eps0, W0a, b0a, W0b, b0b, eps1, W1a, b1a,
           W1b, b1b, eps2, W2a, b2a, W2b, b2b, Wc1, bc1, Wc2, bc2):
    n, in_c = x.shape
    e = edge_index.shape[1]
    g = 128

    tile_rows = _round_up(n // _NTILES + 7, 8)
    n_acc = tile_rows * _NTILES

    ept = -(-e // _NTILES)
    n_batches = _round_up(-(-ept // _EB), 2 * _GB)
    e_pad = _NTILES * n_batches * _EB
    pad = e_pad - e
    src = edge_index[0]
    dst = edge_index[1]
    ar = jnp.arange(pad, dtype=jnp.int32)
    srcp = jnp.concatenate([src, ar % n]).reshape(_NTILES, n_batches, _EB)
    dstp = jnp.concatenate([dst, n + ar % (n_acc - n)]
                           ).reshape(_NTILES, n_batches, _EB)
    zeros = jnp.zeros((n_acc, _FEAT), jnp.float32)

    h = jnp.moveaxis(x.reshape(n, in_c // _FEAT, _FEAT), 1, 0)

    layers = ((eps0, W0a, b0a, W0b, b0b), (eps1, W1a, b1a, W1b, b1b),
              (eps2, W2a, b2a, W2b, b2b))
    nbp = 1000
    batchr = batch.reshape(n // nbp, 1, nbp)
    for li, (eps, wa, ba, wb, bb) in enumerate(layers):
        cin = h.shape[0]
        agg_fn = _make_sc_agg(cin, n, n_acc, tile_rows, n_batches)
        agg = agg_fn(h, srcp, dstp, zeros)
        scale = (1.0 + eps).astype(jnp.float32).reshape(1, 1)
        wa16 = wa.astype(jnp.bfloat16)
        wb16 = wb.astype(jnp.bfloat16)
        if li < len(layers) - 1:
            h = _mlp_call(h, agg, scale, wa16, ba.reshape(1, -1), wb16,
                          bb.reshape(1, -1), nb=1000)
        else:
            return _mlp_pool_cls_call(
                h, agg, scale, wa16, ba.reshape(1, -1), wb16,
                bb.reshape(1, -1), batchr, Wc1.astype(jnp.bfloat16),
                bc1.reshape(1, -1), Wc2.astype(jnp.bfloat16),
                bc2.reshape(1, -1), nbp, g)


# speedup vs baseline: 1.0993x; 1.0051x over previous
import functools

import jax
import jax.numpy as jnp
from jax import lax
from jax.experimental import pallas as pl
from jax.experimental.pallas import tpu as pltpu
from jax.experimental.pallas import tpu_sc as plsc

_FEAT = 128
_NTILES = 16
_EB = 80
_GB = 8
_NBUF = 4


def _round_up(v, m):
    return (v + m - 1) // m * m


def _make_sc_agg(cin, n, n_acc, tile_rows, n_batches):
    n_pairs = cin // 2
    n_groups = n_batches // _GB
    n_blocks = n_batches // (2 * _GB)
    mesh = plsc.VectorSubcoreMesh(core_axis_name="c", subcore_axis_name="s")

    @functools.partial(
        pl.kernel,
        mesh=mesh,
        out_type=jax.ShapeDtypeStruct((cin, n_acc, _FEAT), jnp.float32),
        scratch_types=[
            pltpu.VMEM((2, _GB, _EB), jnp.int32),
            pltpu.VMEM((2, _GB, _EB), jnp.int32),
            pltpu.VMEM((_NBUF, _EB, _FEAT), jnp.float32),
            pltpu.VMEM_SHARED((n_acc, _FEAT), jnp.float32),
            pltpu.SemaphoreType.DMA,
            pltpu.SemaphoreType.DMA,
        ],
    )
    def k(tab, srcr, dstr, zer, out, sidx, didx, rwb, acc, semg, semi):
        c = lax.axis_index("c")
        s = lax.axis_index("s")
        rows = tuple(rwb.at[i] for i in range(_NBUF))

        def idx_pf(goff, p):
            pltpu.async_copy(srcr.at[s].at[pl.ds(goff, _GB)], sidx.at[p],
                             semi)
            pltpu.async_copy(dstr.at[s].at[pl.ds(goff, _GB)], didx.at[p],
                             semi)

        def idx_pf_wait(goff, p):
            pltpu.make_async_copy(srcr.at[s].at[pl.ds(goff, _GB)],
                                  sidx.at[p], semi).wait()
            pltpu.make_async_copy(dstr.at[s].at[pl.ds(goff, _GB)],
                                  didx.at[p], semi).wait()

        def do_block(chunk, g0, full):
            for b in range(16):
                p, rw = b // 8, b % 8
                if b == 0:
                    idx_pf((g0 + 1) * _GB, 1)
                if b == 8 and full:
                    idx_pf((g0 + 2) * _GB, 0)
                if b == 5:
                    idx_pf_wait((g0 + 1) * _GB, 1)
                if b == 13 and full:
                    idx_pf_wait((g0 + 2) * _GB, 0)
                pltpu.make_async_copy(tab.at[chunk].at[sidx.at[p].at[rw]],
                                      rows[b % _NBUF], semg).wait()
                if full or b < 13:
                    if b < 5:
                        np_, nr = 0, b + 3
                    elif b < 13:
                        np_, nr = 1, b - 5
                    else:
                        np_, nr = 0, b - 13
                    pltpu.async_copy(
                        tab.at[chunk].at[sidx.at[np_].at[nr]],
                        rows[(b + 3) % _NBUF], semg)
                pltpu.sync_copy(rows[b % _NBUF], acc.at[didx.at[p].at[rw]],
                                add=True)

        for r in range(n_pairs):
            chunk = 2 * r + c
            row0 = s * tile_rows
            pltpu.sync_copy(srcr.at[s].at[pl.ds(0, _GB)], sidx.at[0])
            pltpu.sync_copy(dstr.at[s].at[pl.ds(0, _GB)], didx.at[0])
            for i in range(3):
                pltpu.async_copy(tab.at[chunk].at[sidx.at[0].at[i]],
                                 rows[i], semg)
            pltpu.sync_copy(zer.at[pl.ds(row0, tile_rows)],
                            acc.at[pl.ds(row0, tile_rows)])
            plsc.subcore_barrier()

            def block_body(t, carry):
                do_block(chunk, 2 * t, True)
                return carry

            lax.fori_loop(0, n_blocks - 1, block_body, 0)
            do_block(chunk, 2 * (n_blocks - 1), False)

            plsc.subcore_barrier()
            pltpu.sync_copy(acc.at[pl.ds(row0, tile_rows)],
                            out.at[chunk].at[pl.ds(row0, tile_rows)])

    return k


def _mlp_call(h, agg, scale, wa, ba, wb, bb, nb):
    cin, n, feat = h.shape
    hdim = wa.shape[1]
    cout = wb.shape[1] // feat
    grid = (n // nb,)

    def body(h_ref, a_ref, sc_ref, wa_ref, ba_ref, wb_ref, bb_ref, out_ref):
        scale_v = sc_ref[0, 0]
        acc = jnp.zeros((nb, hdim), jnp.float32)
        for ci in range(cin):
            z = (scale_v * h_ref[ci].astype(jnp.float32)
                 + a_ref[ci].astype(jnp.float32)).astype(jnp.bfloat16)
            acc = acc + lax.dot_general(
                z, wa_ref[ci * feat:(ci + 1) * feat, :],
                (((1,), (0,)), ((), ())), preferred_element_type=jnp.float32)
        a1 = jnp.maximum(acc + ba_ref[0:1, :], 0.0).astype(jnp.bfloat16)
        h2 = lax.dot_general(a1, wb_ref[...], (((1,), (0,)), ((), ())),
                             preferred_element_type=jnp.float32)
        h2 = jnp.maximum(h2 + bb_ref[0:1, :], 0.0)
        for co in range(cout):
            out_ref[co] = h2[:, co * feat:(co + 1) * feat]

    return pl.pallas_call(
        body,
        grid=grid,
        in_specs=[
            pl.BlockSpec((cin, nb, feat), lambda i: (0, i, 0)),
            pl.BlockSpec((cin, nb, feat), lambda i: (0, i, 0)),
            pl.BlockSpec((1, 1), lambda i: (0, 0)),
            pl.BlockSpec(wa.shape, lambda i: (0, 0)),
            pl.BlockSpec((1, hdim), lambda i: (0, 0)),
            pl.BlockSpec(wb.shape, lambda i: (0, 0)),
            pl.BlockSpec((1, wb.shape[1]), lambda i: (0, 0)),
        ],
        out_specs=pl.BlockSpec((cout, nb, feat), lambda i: (0, i, 0)),
        out_shape=jax.ShapeDtypeStruct((cout, n, feat), jnp.float32),
    )(h, agg, scale, wa, ba, wb, bb)


def _mlp_pool_cls_call(h, agg, scale, wa, ba, wb, bb, batchr, wc1, bc1,
                       wc2, bc2, nb, g):
    cin, n, feat = h.shape
    hdim = wa.shape[1]
    ncls = wc2.shape[1]
    t = n // nb

    def body(h_ref, a_ref, b_ref, sc_ref, wa_ref, ba_ref, wb_ref, bb_ref,
             wc1_ref, bc1_ref, wc2_ref, bc2_ref, out_ref, sums, cnt):
        i = pl.program_id(0)

        @pl.when(i == 0)
        def _():
            sums[...] = jnp.zeros_like(sums)
            cnt[...] = jnp.zeros_like(cnt)

        scale_v = sc_ref[0, 0]
        acc = jnp.zeros((nb, hdim), jnp.float32)
        for ci in range(cin):
            z = (scale_v * h_ref[ci] + a_ref[ci]).astype(jnp.bfloat16)
            acc = acc + lax.dot_general(
                z, wa_ref[ci * feat:(ci + 1) * feat, :],
                (((1,), (0,)), ((), ())), preferred_element_type=jnp.float32)
        a1 = jnp.maximum(acc + ba_ref[0:1, :], 0.0).astype(jnp.bfloat16)
        h2 = lax.dot_general(a1, wb_ref[...], (((1,), (0,)), ((), ())),
                             preferred_element_type=jnp.float32)
        h2 = jnp.maximum(h2 + bb_ref[0:1, :], 0.0).astype(jnp.bfloat16)

        b2 = b_ref[0]
        iota_g = lax.broadcasted_iota(jnp.int32, (g, nb), 0)
        oht = (b2 == iota_g).astype(jnp.bfloat16)
        sums[...] += lax.dot_general(oht, h2, (((1,), (0,)), ((), ())),
                                     preferred_element_type=jnp.float32)
        cnt[...] += lax.dot_general(
            oht, jnp.ones((nb, feat), jnp.bfloat16),
            (((1,), (0,)), ((), ())), preferred_element_type=jnp.float32)

        @pl.when(i == t - 1)
        def _():
            rcp = 1.0 / jnp.maximum(cnt[...], 1.0)
            ac2 = jnp.zeros((g, hdim), jnp.float32)
            for ci in range(hdim // feat):
                pooled = (sums[:, ci * feat:(ci + 1) * feat] * rcp
                          ).astype(jnp.bfloat16)
                ac2 = ac2 + lax.dot_general(
                    pooled, wc1_ref[ci * feat:(ci + 1) * feat, :],
                    (((1,), (0,)), ((), ())),
                    preferred_element_type=jnp.float32)
            hc = jnp.maximum(ac2 + bc1_ref[0:1, :], 0.0).astype(jnp.bfloat16)
            logits = lax.dot_general(hc, wc2_ref[...],
                                     (((1,), (0,)), ((), ())),
                                     preferred_element_type=jnp.float32)
            out_ref[...] = logits + bc2_ref[0:1, :]

    return pl.pallas_call(
        body,
        grid=(t,),
        in_specs=[
            pl.BlockSpec((cin, nb, feat), lambda i: (0, i, 0)),
            pl.BlockSpec((cin, nb, feat), lambda i: (0, i, 0)),
            pl.BlockSpec((1, 1, nb), lambda i: (i, 0, 0)),
            pl.BlockSpec((1, 1), lambda i: (0, 0)),
            pl.BlockSpec(wa.shape, lambda i: (0, 0)),
            pl.BlockSpec((1, hdim), lambda i: (0, 0)),
            pl.BlockSpec(wb.shape, lambda i: (0, 0)),
            pl.BlockSpec((1, wb.shape[1]), lambda i: (0, 0)),
            pl.BlockSpec(wc1.shape, lambda i: (0, 0)),
            pl.BlockSpec((1, wc1.shape[1]), lambda i: (0, 0)),
            pl.BlockSpec(wc2.shape, lambda i: (0, 0)),
            pl.BlockSpec((1, ncls), lambda i: (0, 0)),
        ],
        out_specs=pl.BlockSpec((g, ncls), lambda i: (0, 0)),
        out_shape=jax.ShapeDtypeStruct((g, ncls), jnp.float32),
        scratch_shapes=[
            pltpu.VMEM((g, hdim), jnp.float32),
            pltpu.VMEM((g, feat), jnp.float32),
        ],
    )(h, agg, batchr, scale, wa, ba, wb, bb, wc1, bc1, wc2, bc2)


def _pool_cls_call(h, batchr, wc1, bc1, wc2, bc2, nbp, g):
    cin, n, feat = h.shape
    hdim = wc1.shape[1]
    ncls = wc2.shape[1]
    t = n // nbp

    def body(h_ref, b_ref, wc1_ref, bc1_ref, wc2_ref, bc2_ref, out_ref,
             sums, cnt):
        i = pl.program_id(0)

        @pl.when(i == 0)
        def _():
            sums[...] = jnp.zeros_like(sums)
            cnt[...] = jnp.zeros_like(cnt)

        b2 = b_ref[0]
        iota_g = lax.broadcasted_iota(jnp.int32, (g, nbp), 0)
        oht = (b2 == iota_g).astype(jnp.bfloat16)
        for ci in range(cin):
            sums[ci] += lax.dot_general(
                oht, h_ref[ci].astype(jnp.bfloat16), (((1,), (0,)), ((), ())),
                preferred_element_type=jnp.float32)
        cnt[...] += lax.dot_general(
            oht, jnp.ones((nbp, feat), jnp.bfloat16),
            (((1,), (0,)), ((), ())), preferred_element_type=jnp.float32)

        @pl.when(i == t - 1)
        def _():
            rcp = 1.0 / jnp.maximum(cnt[...], 1.0)
            acc = jnp.zeros((g, hdim), jnp.float32)
            for ci in range(cin):
                pooled = (sums[ci] * rcp).astype(jnp.bfloat16)
                acc = acc + lax.dot_general(
                    pooled, wc1_ref[ci * feat:(ci + 1) * feat, :],
                    (((1,), (0,)), ((), ())),
                    preferred_element_type=jnp.float32)
            hc = jnp.maximum(acc + bc1_ref[0:1, :], 0.0).astype(jnp.bfloat16)
            logits = lax.dot_general(hc, wc2_ref[...],
                                     (((1,), (0,)), ((), ())),
                                     preferred_element_type=jnp.float32)
            out_ref[...] = logits + bc2_ref[0:1, :]

    return pl.pallas_call(
        body,
        grid=(t,),
        in_specs=[
            pl.BlockSpec((cin, nbp, feat), lambda i: (0, i, 0)),
            pl.BlockSpec((1, 1, nbp), lambda i: (i, 0, 0)),
            pl.BlockSpec(wc1.shape, lambda i: (0, 0)),
            pl.BlockSpec((1, hdim), lambda i: (0, 0)),
            pl.BlockSpec(wc2.shape, lambda i: (0, 0)),
            pl.BlockSpec((1, ncls), lambda i: (0, 0)),
        ],
        out_specs=pl.BlockSpec((g, ncls), lambda i: (0, 0)),
        out_shape=jax.ShapeDtypeStruct((g, ncls), jnp.float32),
        scratch_shapes=[
            pltpu.VMEM((cin, g, feat), jnp.float32),
            pltpu.VMEM((g, feat), jnp.float32),
        ],
    )(h, batchr, wc1, bc1, wc2, bc2)


def kernel(x, edge_index, batch, eps0, W0a, b0a, W0b, b0b, eps1, W1a, b1a,
           W1b, b1b, eps2, W2a, b2a, W2b, b2b, Wc1, bc1, Wc2, bc2):
    n, in_c = x.shape
    e = edge_index.shape[1]
    g = 128

    tile_rows = _round_up(n // _NTILES + 7, 8)
    n_acc = tile_rows * _NTILES

    ept = -(-e // _NTILES)
    n_batches = _round_up(-(-ept // _EB), 2 * _GB)
    e_pad = _NTILES * n_batches * _EB
    pad = e_pad - e
    src = edge_index[0]
    dst = edge_index[1]
    ar = jnp.arange(pad, dtype=jnp.int32)
    srcp = jnp.concatenate([src, ar % n]).reshape(_NTILES, n_batches, _EB)
    dstp = jnp.concatenate([dst, n + ar % (n_acc - n)]
                           ).reshape(_NTILES, n_batches, _EB)
    zeros = jnp.zeros((n_acc, _FEAT), jnp.float32)

    h = jnp.moveaxis(x.reshape(n, in_c // _FEAT, _FEAT), 1, 0)

    layers = ((eps0, W0a, b0a, W0b, b0b), (eps1, W1a, b1a, W1b, b1b),
              (eps2, W2a, b2a, W2b, b2b))
    nbp = 1000
    batchr = batch.reshape(n // nbp, 1, nbp)
    for li, (eps, wa, ba, wb, bb) in enumerate(layers):
        cin = h.shape[0]
        agg_fn = _make_sc_agg(cin, n, n_acc, tile_rows, n_batches)
        agg = agg_fn(h, srcp, dstp, zeros)
        scale = (1.0 + eps).astype(jnp.float32).reshape(1, 1)
        wa16 = wa.astype(jnp.bfloat16)
        wb16 = wb.astype(jnp.bfloat16)
        if li < len(layers) - 1:
            h = _mlp_call(h, agg, scale, wa16, ba.reshape(1, -1), wb16,
                          bb.reshape(1, -1), nb=2000)
        else:
            return _mlp_pool_cls_call(
                h, agg, scale, wa16, ba.reshape(1, -1), wb16,
                bb.reshape(1, -1), batchr, Wc1.astype(jnp.bfloat16),
                bc1.reshape(1, -1), Wc2.astype(jnp.bfloat16),
                bc2.reshape(1, -1), nbp, g)


# speedup vs baseline: 1.0995x; 1.0002x over previous
import functools

import jax
import jax.numpy as jnp
from jax import lax
from jax.experimental import pallas as pl
from jax.experimental.pallas import tpu as pltpu
from jax.experimental.pallas import tpu_sc as plsc

_FEAT = 128
_NTILES = 16
_EB = 80
_GB = 8
_NBUF = 4


def _round_up(v, m):
    return (v + m - 1) // m * m


def _make_sc_agg(cin, n, n_acc, tile_rows, n_batches):
    n_pairs = cin // 2
    n_groups = n_batches // _GB
    n_blocks = n_batches // (2 * _GB)
    mesh = plsc.VectorSubcoreMesh(core_axis_name="c", subcore_axis_name="s")

    @functools.partial(
        pl.kernel,
        mesh=mesh,
        out_type=jax.ShapeDtypeStruct((cin, n_acc, _FEAT), jnp.float32),
        scratch_types=[
            pltpu.VMEM((2, _GB, _EB), jnp.int32),
            pltpu.VMEM((2, _GB, _EB), jnp.int32),
            pltpu.VMEM((_NBUF, _EB, _FEAT), jnp.float32),
            pltpu.VMEM_SHARED((n_acc, _FEAT), jnp.float32),
            pltpu.SemaphoreType.DMA,
            pltpu.SemaphoreType.DMA,
        ],
    )
    def k(tab, srcr, dstr, zer, out, sidx, didx, rwb, acc, semg, semi):
        c = lax.axis_index("c")
        s = lax.axis_index("s")
        rows = tuple(rwb.at[i] for i in range(_NBUF))

        def idx_pf(goff, p):
            pltpu.async_copy(srcr.at[s].at[pl.ds(goff, _GB)], sidx.at[p],
                             semi)
            pltpu.async_copy(dstr.at[s].at[pl.ds(goff, _GB)], didx.at[p],
                             semi)

        def idx_pf_wait(goff, p):
            pltpu.make_async_copy(srcr.at[s].at[pl.ds(goff, _GB)],
                                  sidx.at[p], semi).wait()
            pltpu.make_async_copy(dstr.at[s].at[pl.ds(goff, _GB)],
                                  didx.at[p], semi).wait()

        def do_block(chunk, g0, full):
            for b in range(16):
                p, rw = b // 8, b % 8
                if b == 0:
                    idx_pf((g0 + 1) * _GB, 1)
                if b == 8 and full:
                    idx_pf((g0 + 2) * _GB, 0)
                if b == 5:
                    idx_pf_wait((g0 + 1) * _GB, 1)
                if b == 13 and full:
                    idx_pf_wait((g0 + 2) * _GB, 0)
                pltpu.make_async_copy(tab.at[chunk].at[sidx.at[p].at[rw]],
                                      rows[b % _NBUF], semg).wait()
                if full or b < 13:
                    if b < 5:
                        np_, nr = 0, b + 3
                    elif b < 13:
                        np_, nr = 1, b - 5
                    else:
                        np_, nr = 0, b - 13
                    pltpu.async_copy(
                        tab.at[chunk].at[sidx.at[np_].at[nr]],
                        rows[(b + 3) % _NBUF], semg)
                pltpu.sync_copy(rows[b % _NBUF], acc.at[didx.at[p].at[rw]],
                                add=True)

        for r in range(n_pairs):
            chunk = 2 * r + c
            row0 = s * tile_rows
            pltpu.sync_copy(srcr.at[s].at[pl.ds(0, _GB)], sidx.at[0])
            pltpu.sync_copy(dstr.at[s].at[pl.ds(0, _GB)], didx.at[0])
            for i in range(3):
                pltpu.async_copy(tab.at[chunk].at[sidx.at[0].at[i]],
                                 rows[i], semg)
            pltpu.sync_copy(zer.at[pl.ds(row0, tile_rows)],
                            acc.at[pl.ds(row0, tile_rows)])
            plsc.subcore_barrier()

            def block_body(t, carry):
                do_block(chunk, 2 * t, True)
                return carry

            lax.fori_loop(0, n_blocks - 1, block_body, 0)
            do_block(chunk, 2 * (n_blocks - 1), False)

            plsc.subcore_barrier()
            pltpu.sync_copy(acc.at[pl.ds(row0, tile_rows)],
                            out.at[chunk].at[pl.ds(row0, tile_rows)])

    return k


def _mlp_call(h, agg, scale, wa, ba, wb, bb, nb):
    cin, n, feat = h.shape
    hdim = wa.shape[1]
    cout = wb.shape[1] // feat
    grid = (n // nb,)

    def body(h_ref, a_ref, sc_ref, wa_ref, ba_ref, wb_ref, bb_ref, out_ref):
        scale_v = sc_ref[0, 0]
        acc = jnp.zeros((nb, hdim), jnp.float32)
        for ci in range(cin):
            z = (scale_v * h_ref[ci].astype(jnp.float32)
                 + a_ref[ci].astype(jnp.float32)).astype(jnp.bfloat16)
            acc = acc + lax.dot_general(
                z, wa_ref[ci * feat:(ci + 1) * feat, :],
                (((1,), (0,)), ((), ())), preferred_element_type=jnp.float32)
        a1 = jnp.maximum(acc + ba_ref[0:1, :], 0.0).astype(jnp.bfloat16)
        h2 = lax.dot_general(a1, wb_ref[...], (((1,), (0,)), ((), ())),
                             preferred_element_type=jnp.float32)
        h2 = jnp.maximum(h2 + bb_ref[0:1, :], 0.0)
        for co in range(cout):
            out_ref[co] = h2[:, co * feat:(co + 1) * feat]

    return pl.pallas_call(
        body,
        grid=grid,
        in_specs=[
            pl.BlockSpec((cin, nb, feat), lambda i: (0, i, 0)),
            pl.BlockSpec((cin, nb, feat), lambda i: (0, i, 0)),
            pl.BlockSpec((1, 1), lambda i: (0, 0)),
            pl.BlockSpec(wa.shape, lambda i: (0, 0)),
            pl.BlockSpec((1, hdim), lambda i: (0, 0)),
            pl.BlockSpec(wb.shape, lambda i: (0, 0)),
            pl.BlockSpec((1, wb.shape[1]), lambda i: (0, 0)),
        ],
        out_specs=pl.BlockSpec((cout, nb, feat), lambda i: (0, i, 0)),
        out_shape=jax.ShapeDtypeStruct((cout, n, feat), jnp.float32),
    )(h, agg, scale, wa, ba, wb, bb)


def _mlp_pool_cls_call(h, agg, scale, wa, ba, wb, bb, batchr, wc1, bc1,
                       wc2, bc2, nb, g):
    cin, n, feat = h.shape
    hdim = wa.shape[1]
    ncls = wc2.shape[1]
    t = n // nb

    def body(h_ref, a_ref, b_ref, sc_ref, wa_ref, ba_ref, wb_ref, bb_ref,
             wc1_ref, bc1_ref, wc2_ref, bc2_ref, out_ref, sums, cnt):
        i = pl.program_id(0)

        @pl.when(i == 0)
        def _():
            sums[...] = jnp.zeros_like(sums)
            cnt[...] = jnp.zeros_like(cnt)

        scale_v = sc_ref[0, 0]
        acc = jnp.zeros((nb, hdim), jnp.float32)
        for ci in range(cin):
            z = (scale_v * h_ref[ci] + a_ref[ci]).astype(jnp.bfloat16)
            acc = acc + lax.dot_general(
                z, wa_ref[ci * feat:(ci + 1) * feat, :],
                (((1,), (0,)), ((), ())), preferred_element_type=jnp.float32)
        a1 = jnp.maximum(acc + ba_ref[0:1, :], 0.0).astype(jnp.bfloat16)
        h2 = lax.dot_general(a1, wb_ref[...], (((1,), (0,)), ((), ())),
                             preferred_element_type=jnp.float32)
        h2 = jnp.maximum(h2 + bb_ref[0:1, :], 0.0).astype(jnp.bfloat16)

        b2 = b_ref[0]
        iota_g = lax.broadcasted_iota(jnp.int32, (g, nb), 0)
        oht = (b2 == iota_g).astype(jnp.bfloat16)
        sums[...] += lax.dot_general(oht, h2, (((1,), (0,)), ((), ())),
                                     preferred_element_type=jnp.float32)
        cnt[...] += lax.dot_general(
            oht, jnp.ones((nb, feat), jnp.bfloat16),
            (((1,), (0,)), ((), ())), preferred_element_type=jnp.float32)

        @pl.when(i == t - 1)
        def _():
            rcp = 1.0 / jnp.maximum(cnt[...], 1.0)
            ac2 = jnp.zeros((g, hdim), jnp.float32)
            for ci in range(hdim // feat):
                pooled = (sums[:, ci * feat:(ci + 1) * feat] * rcp
                          ).astype(jnp.bfloat16)
                ac2 = ac2 + lax.dot_general(
                    pooled, wc1_ref[ci * feat:(ci + 1) * feat, :],
                    (((1,), (0,)), ((), ())),
                    preferred_element_type=jnp.float32)
            hc = jnp.maximum(ac2 + bc1_ref[0:1, :], 0.0).astype(jnp.bfloat16)
            logits = lax.dot_general(hc, wc2_ref[...],
                                     (((1,), (0,)), ((), ())),
                                     preferred_element_type=jnp.float32)
            out_ref[...] = logits + bc2_ref[0:1, :]

    return pl.pallas_call(
        body,
        grid=(t,),
        in_specs=[
            pl.BlockSpec((cin, nb, feat), lambda i: (0, i, 0)),
            pl.BlockSpec((cin, nb, feat), lambda i: (0, i, 0)),
            pl.BlockSpec((1, 1, nb), lambda i: (i, 0, 0)),
            pl.BlockSpec((1, 1), lambda i: (0, 0)),
            pl.BlockSpec(wa.shape, lambda i: (0, 0)),
            pl.BlockSpec((1, hdim), lambda i: (0, 0)),
            pl.BlockSpec(wb.shape, lambda i: (0, 0)),
            pl.BlockSpec((1, wb.shape[1]), lambda i: (0, 0)),
            pl.BlockSpec(wc1.shape, lambda i: (0, 0)),
            pl.BlockSpec((1, wc1.shape[1]), lambda i: (0, 0)),
            pl.BlockSpec(wc2.shape, lambda i: (0, 0)),
            pl.BlockSpec((1, ncls), lambda i: (0, 0)),
        ],
        out_specs=pl.BlockSpec((g, ncls), lambda i: (0, 0)),
        out_shape=jax.ShapeDtypeStruct((g, ncls), jnp.float32),
        scratch_shapes=[
            pltpu.VMEM((g, hdim), jnp.float32),
            pltpu.VMEM((g, feat), jnp.float32),
        ],
    )(h, agg, batchr, scale, wa, ba, wb, bb, wc1, bc1, wc2, bc2)


def _pool_cls_call(h, batchr, wc1, bc1, wc2, bc2, nbp, g):
    cin, n, feat = h.shape
    hdim = wc1.shape[1]
    ncls = wc2.shape[1]
    t = n // nbp

    def body(h_ref, b_ref, wc1_ref, bc1_ref, wc2_ref, bc2_ref, out_ref,
             sums, cnt):
        i = pl.program_id(0)

        @pl.when(i == 0)
        def _():
            sums[...] = jnp.zeros_like(sums)
            cnt[...] = jnp.zeros_like(cnt)

        b2 = b_ref[0]
        iota_g = lax.broadcasted_iota(jnp.int32, (g, nbp), 0)
        oht = (b2 == iota_g).astype(jnp.bfloat16)
        for ci in range(cin):
            sums[ci] += lax.dot_general(
                oht, h_ref[ci].astype(jnp.bfloat16), (((1,), (0,)), ((), ())),
                preferred_element_type=jnp.float32)
        cnt[...] += lax.dot_general(
            oht, jnp.ones((nbp, feat), jnp.bfloat16),
            (((1,), (0,)), ((), ())), preferred_element_type=jnp.float32)

        @pl.when(i == t - 1)
        def _():
            rcp = 1.0 / jnp.maximum(cnt[...], 1.0)
            acc = jnp.zeros((g, hdim), jnp.float32)
            for ci in range(cin):
                pooled = (sums[ci] * rcp).astype(jnp.bfloat16)
                acc = acc + lax.dot_general(
                    pooled, wc1_ref[ci * feat:(ci + 1) * feat, :],
                    (((1,), (0,)), ((), ())),
                    preferred_element_type=jnp.float32)
            hc = jnp.maximum(acc + bc1_ref[0:1, :], 0.0).astype(jnp.bfloat16)
            logits = lax.dot_general(hc, wc2_ref[...],
                                     (((1,), (0,)), ((), ())),
                                     preferred_element_type=jnp.float32)
            out_ref[...] = logits + bc2_ref[0:1, :]

    return pl.pallas_call(
        body,
        grid=(t,),
        in_specs=[
            pl.BlockSpec((cin, nbp, feat), lambda i: (0, i, 0)),
            pl.BlockSpec((1, 1, nbp), lambda i: (i, 0, 0)),
            pl.BlockSpec(wc1.shape, lambda i: (0, 0)),
            pl.BlockSpec((1, hdim), lambda i: (0, 0)),
            pl.BlockSpec(wc2.shape, lambda i: (0, 0)),
            pl.BlockSpec((1, ncls), lambda i: (0, 0)),
        ],
        out_specs=pl.BlockSpec((g, ncls), lambda i: (0, 0)),
        out_shape=jax.ShapeDtypeStruct((g, ncls), jnp.float32),
        scratch_shapes=[
            pltpu.VMEM((cin, g, feat), jnp.float32),
            pltpu.VMEM((g, feat), jnp.float32),
        ],
    )(h, batchr, wc1, bc1, wc2, bc2)


def kernel(x, edge_index, batch, eps0, W0a, b0a, W0b, b0b, eps1, W1a, b1a,
           W1b, b1b, eps2, W2a, b2a, W2b, b2b, Wc1, bc1, Wc2, bc2):
    n, in_c = x.shape
    e = edge_index.shape[1]
    g = 128

    tile_rows = _round_up(n // _NTILES + 7, 8)
    n_acc = tile_rows * _NTILES

    ept = -(-e // _NTILES)
    n_batches = _round_up(-(-ept // _EB), 2 * _GB)
    e_pad = _NTILES * n_batches * _EB
    pad = e_pad - e
    src = edge_index[0]
    dst = edge_index[1]
    ar = jnp.arange(pad, dtype=jnp.int32)
    srcp = jnp.concatenate([src, ar % n]).reshape(_NTILES, n_batches, _EB)
    dstp = jnp.concatenate([dst, n + ar % (n_acc - n)]
                           ).reshape(_NTILES, n_batches, _EB)
    zeros = jnp.zeros((n_acc, _FEAT), jnp.float32)

    h = jnp.moveaxis(x.reshape(n, in_c // _FEAT, _FEAT), 1, 0)

    layers = ((eps0, W0a, b0a, W0b, b0b), (eps1, W1a, b1a, W1b, b1b),
              (eps2, W2a, b2a, W2b, b2b))
    nbp = 2000
    batchr = batch.reshape(n // nbp, 1, nbp)
    for li, (eps, wa, ba, wb, bb) in enumerate(layers):
        cin = h.shape[0]
        agg_fn = _make_sc_agg(cin, n, n_acc, tile_rows, n_batches)
        agg = agg_fn(h, srcp, dstp, zeros)
        scale = (1.0 + eps).astype(jnp.float32).reshape(1, 1)
        wa16 = wa.astype(jnp.bfloat16)
        wb16 = wb.astype(jnp.bfloat16)
        if li < len(layers) - 1:
            h = _mlp_call(h, agg, scale, wa16, ba.reshape(1, -1), wb16,
                          bb.reshape(1, -1), nb=2000)
        else:
            return _mlp_pool_cls_call(
                h, agg, scale, wa16, ba.reshape(1, -1), wb16,
                bb.reshape(1, -1), batchr, Wc1.astype(jnp.bfloat16),
                bc1.reshape(1, -1), Wc2.astype(jnp.bfloat16),
                bc2.reshape(1, -1), nbp, g)
